# Initial kernel scaffold; baseline (speedup 1.0000x reference)
import jax
import jax.numpy as jnp
from jax.experimental import pallas as pl


def kernel(x, edge_index, edge_attr, batch_index, mol_features, params):
    raise NotImplementedError("write your pallas kernel here")


# speedup vs baseline: 9.9125x; 9.9125x over previous
import functools

import jax
import jax.numpy as jnp
from jax import lax
from jax.experimental import pallas as pl
from jax.experimental.pallas import tpu as pltpu
from jax.experimental.pallas import tpu_sc as plsc

F32 = jnp.float32
I32 = jnp.int32

N_NODES = 10000
N_EDGES = 320000
D = 128
EDGE_DIM = 16
EDGE_HID = 32
N_GRAPHS = 64

NC, NS, LANES = 2, 16, 16
NWORK = NC * NS
N_PAD = 10240
NSEG = N_PAD // NS
E_PAD = 327680
EPT = E_PAD // NWORK
W = 128
NWIN = EPT // W

EB = 8192
NB = 2048
NBLK = N_PAD // NB

@functools.cache
def _sc_mesh():
    return plsc.VectorSubcoreMesh(
        core_axis_name="c", subcore_axis_name="s",
        num_cores=NC, num_subcores=NS)


def _edge_mlp_body(ea_ref, w1_ref, b1_ref, w2_ref, b2_ref, out_ref):
    i = pl.program_id(0)
    h = jnp.dot(w1_ref[...], ea_ref[...], preferred_element_type=F32)
    h = jnp.maximum(h + b1_ref[...], 0.0)
    z = jnp.dot(w2_ref[...], h, preferred_element_type=F32) + b2_ref[...]
    e = jax.nn.sigmoid(z)
    pos = i * EB + lax.broadcasted_iota(I32, (8, EB), 1)
    out_ref[...] = jnp.where(pos < N_EDGES, e, 0.0)


def _edge_mlp(eaT, w1t, b1c, w2t, b2c):
    return pl.pallas_call(
        _edge_mlp_body,
        grid=(E_PAD // EB,),
        in_specs=[
            pl.BlockSpec((16, EB), lambda i: (0, i)),
            pl.BlockSpec((96, 16), lambda i: (0, 0)),
            pl.BlockSpec((96, 1), lambda i: (0, 0)),
            pl.BlockSpec((8, 96), lambda i: (0, 0)),
            pl.BlockSpec((8, 1), lambda i: (0, 0)),
        ],
        out_specs=pl.BlockSpec((8, EB), lambda i: (0, i)),
        out_shape=jax.ShapeDtypeStruct((8, E_PAD), F32),
    )(eaT, w1t, b1c, w2t, b2c)


@functools.cache
def _build_deg_kernel():
    return functools.partial(
        pl.kernel,
        out_type=jax.ShapeDtypeStruct((NC * 3 * N_PAD,), F32),
        mesh=_sc_mesh(),
        compiler_params=pltpu.CompilerParams(needs_layout_passes=False),
        scratch_types=[
            pltpu.VMEM((NWIN, W), I32),
            pltpu.VMEM((EPT,), F32),
            pltpu.VMEM_SHARED((N_PAD,), F32),
            pltpu.VMEM_SHARED((N_PAD,), F32),
            pltpu.VMEM_SHARED((N_PAD,), F32),
        ],
    )(_deg_body)


def _deg_body(col2_hbm, e_hbm, zn_hbm, out_hbm, col_loc, e_loc,
              deg0, deg1, deg2):
    c = lax.axis_index("c")
    s = lax.axis_index("s")
    wid = c * NS + s
    base = wid * EPT
    pltpu.sync_copy(col2_hbm.at[wid], col_loc)
    for dref in (deg0, deg1, deg2):
        pltpu.sync_copy(zn_hbm.at[pl.ds(s * NSEG, NSEG)],
                        dref.at[pl.ds(s * NSEG, NSEG)])
    plsc.subcore_barrier()
    for l, dref in enumerate((deg0, deg1, deg2)):
        pltpu.sync_copy(e_hbm.at[pl.ds(l * E_PAD + base, EPT)], e_loc)

        def win_body(wn, carry, dref=dref):
            pltpu.sync_copy(e_loc.at[pl.ds(wn * W, W)],
                            dref.at[col_loc.at[wn]], add=True)
            return carry

        lax.fori_loop(0, NWIN, win_body, 0)
    plsc.subcore_barrier()

    @pl.when(s == 0)
    def _():
        for l, dref in enumerate((deg0, deg1, deg2)):
            pltpu.sync_copy(dref, out_hbm.at[pl.ds((c * 3 + l) * N_PAD,
                                                   N_PAD)])


def _col_of(dinv, l):
    sel = (lax.broadcasted_iota(I32, (3, 1), 0) == l).astype(F32)
    return lax.dot_general(dinv, sel, (((0,), (0,)), ((), ())),
                           preferred_element_type=F32)


def _norm_body(degp_ref, x_ref, w0_ref, dinv_ref, yt_ref):
    dp = degp_ref[...]
    deg = dp[:3] + dp[3:] + 1.0
    dinv = lax.rsqrt(jnp.maximum(deg, 1e-12))
    dinv_ref[...] = dinv
    xt = jnp.dot(x_ref[...], w0_ref[...], preferred_element_type=F32)
    yt_ref[...] = xt * _col_of(dinv, 0)


def _norm_xt0(degp, x_p, w0):
    return pl.pallas_call(
        _norm_body,
        grid=(NBLK,),
        in_specs=[
            pl.BlockSpec((NC * 3, NB), lambda i: (0, i)),
            pl.BlockSpec((NB, D), lambda i: (i, 0)),
            pl.BlockSpec((D, D), lambda i: (0, 0)),
        ],
        out_specs=[
            pl.BlockSpec((3, NB), lambda i: (0, i)),
            pl.BlockSpec((NB, D), lambda i: (i, 0)),
        ],
        out_shape=[
            jax.ShapeDtypeStruct((3, N_PAD), F32),
            jax.ShapeDtypeStruct((N_PAD, D), F32),
        ],
    )(degp, x_p, w0)


@functools.cache
def _build_agg_kernel():
    return functools.partial(
        pl.kernel,
        out_type=jax.ShapeDtypeStruct((NC, N_PAD, D), F32),
        mesh=_sc_mesh(),
        compiler_params=pltpu.CompilerParams(needs_layout_passes=False),
        scratch_types=[
            pltpu.VMEM((EPT,), I32),
            pltpu.VMEM((NWIN, W), I32),
            pltpu.VMEM((EPT + LANES,), F32),
            pltpu.VMEM((W, D), F32),
            pltpu.VMEM_SHARED((N_PAD, D), F32),
            pltpu.SemaphoreType.DMA,
        ],
    )(_agg_body)


def _agg_body(yt_hbm, row_hbm, col2_hbm, e_hbm, zf_hbm,
              out_hbm, row_loc, col_loc, e_loc, rows_v, acc, sem):
    c = lax.axis_index("c")
    s = lax.axis_index("s")
    wid = c * NS + s
    base = wid * EPT
    pltpu.sync_copy(row_hbm.at[pl.ds(base, EPT)], row_loc)
    pltpu.sync_copy(col2_hbm.at[wid], col_loc)
    pltpu.sync_copy(e_hbm.at[pl.ds(base, EPT)],
                    e_loc.at[pl.ds(0, EPT)])

    @pl.when(c == 0)
    def _():
        pltpu.sync_copy(yt_hbm.at[pl.ds(s * NSEG, NSEG)],
                        acc.at[pl.ds(s * NSEG, NSEG)])

    @pl.when(c != 0)
    def _():
        pltpu.sync_copy(zf_hbm.at[pl.ds(s * NSEG, NSEG)],
                        acc.at[pl.ds(s * NSEG, NSEG)])

    plsc.subcore_barrier()

    def win_body(wn, carry):
        pltpu.async_copy(yt_hbm.at[row_loc.at[pl.ds(wn * W, W)]],
                         rows_v, sem).wait()

        def edge_body(i, carry2):
            w16 = jnp.full((LANES,), e_loc[pl.ds(wn * W + i, LANES)][0], F32)
            for jj in range(D // LANES):
                v = rows_v[i, pl.ds(jj * LANES, LANES)]
                rows_v[i, pl.ds(jj * LANES, LANES)] = v * w16
            return carry2

        lax.fori_loop(0, W, edge_body, 0)
        pltpu.sync_copy(rows_v, acc.at[col_loc.at[wn]], add=True)
        return carry

    lax.fori_loop(0, NWIN, win_body, 0)
    plsc.subcore_barrier()
    pltpu.sync_copy(acc.at[pl.ds(s * NSEG, NSEG)],
                    out_hbm.at[c, pl.ds(s * NSEG, NSEG)])


def _post_body(l, part_ref, bias_ref, gw_ref, gb_ref, lwn_ref, dinv_ref,
               yt_ref):
    p = part_ref[...]
    dv = dinv_ref[...]
    agg = (p[0] + p[1]) * _col_of(dv, l) + bias_ref[...]
    y = jnp.maximum(jnp.dot(agg, gw_ref[...], preferred_element_type=F32)
                    + gb_ref[...], 0.0)
    xt = jnp.dot(y, lwn_ref[...], preferred_element_type=F32)
    yt_ref[...] = xt * _col_of(dv, l + 1)


def _post(l, part, bias, gw, gb, lwn, dinv3):
    return pl.pallas_call(
        functools.partial(_post_body, l),
        grid=(NBLK,),
        in_specs=[
            pl.BlockSpec((NC, NB, D), lambda i: (0, i, 0)),
            pl.BlockSpec((1, D), lambda i: (0, 0)),
            pl.BlockSpec((D, D), lambda i: (0, 0)),
            pl.BlockSpec((1, D), lambda i: (0, 0)),
            pl.BlockSpec((D, D), lambda i: (0, 0)),
            pl.BlockSpec((3, NB), lambda i: (0, i)),
        ],
        out_specs=pl.BlockSpec((NB, D), lambda i: (i, 0)),
        out_shape=jax.ShapeDtypeStruct((N_PAD, D), F32),
    )(part, bias, gw, gb, lwn, dinv3)


def _post_last_body(part_ref, bias_ref, gw_ref, gb_ref, dinv_ref, y_ref):
    p = part_ref[...]
    agg = (p[0] + p[1]) * _col_of(dinv_ref[...], 2) + bias_ref[...]
    y_ref[...] = jnp.maximum(
        jnp.dot(agg, gw_ref[...], preferred_element_type=F32) + gb_ref[...],
        0.0)


def _post_last(part, bias, gw, gb, dinv3):
    return pl.pallas_call(
        _post_last_body,
        grid=(NBLK,),
        in_specs=[
            pl.BlockSpec((NC, NB, D), lambda i: (0, i, 0)),
            pl.BlockSpec((1, D), lambda i: (0, 0)),
            pl.BlockSpec((D, D), lambda i: (0, 0)),
            pl.BlockSpec((1, D), lambda i: (0, 0)),
            pl.BlockSpec((3, NB), lambda i: (0, i)),
        ],
        out_specs=pl.BlockSpec((NB, D), lambda i: (i, 0)),
        out_shape=jax.ShapeDtypeStruct((N_PAD, D), F32),
    )(part, bias, gw, gb, dinv3)


def _pool_body(bi_ref, y_ref, sums_ref, cnt_ref):
    i = pl.program_id(0)

    @pl.when(i == 0)
    def _():
        sums_ref[...] = jnp.zeros((N_GRAPHS, D), F32)
        cnt_ref[...] = jnp.zeros((N_GRAPHS, 1), F32)

    b = bi_ref[...].reshape(1, NB)
    oh = (lax.broadcasted_iota(I32, (N_GRAPHS, NB), 0) == b).astype(F32)
    sums_ref[...] += jnp.dot(oh, y_ref[...], preferred_element_type=F32)
    cnt_ref[...] += jnp.sum(oh, axis=1, keepdims=True)


def _pool(bi3, y2):
    return pl.pallas_call(
        _pool_body,
        grid=(NBLK,),
        in_specs=[
            pl.BlockSpec((1, 1, NB), lambda i: (i, 0, 0)),
            pl.BlockSpec((NB, D), lambda i: (i, 0)),
        ],
        out_specs=[
            pl.BlockSpec((N_GRAPHS, D), lambda i: (0, 0)),
            pl.BlockSpec((N_GRAPHS, 1), lambda i: (0, 0)),
        ],
        out_shape=[
            jax.ShapeDtypeStruct((N_GRAPHS, D), F32),
            jax.ShapeDtypeStruct((N_GRAPHS, 1), F32),
        ],
    )(bi3, y2)


def _head_body(sums_ref, cnt_ref, mol_ref, m0_ref, m0b_ref, m1_ref, m1b_ref,
               m2_ref, m2b_ref, pa_ref, pb_ref, p0b_ref, p1_ref, p1b_ref,
               ow_ref, ob_ref, out_ref):
    pooled = sums_ref[...] / jnp.maximum(cnt_ref[...], 1.0)
    h2 = jnp.maximum(jnp.dot(mol_ref[...], m0_ref[...],
                             preferred_element_type=F32) + m0b_ref[...], 0.0)
    h2 = jnp.maximum(jnp.dot(h2, m1_ref[...],
                             preferred_element_type=F32) + m1b_ref[...], 0.0)
    h2 = jnp.maximum(jnp.dot(h2, m2_ref[...],
                             preferred_element_type=F32) + m2b_ref[...], 0.0)
    h = jnp.maximum(jnp.dot(pooled, pa_ref[...], preferred_element_type=F32)
                    + jnp.dot(h2, pb_ref[...], preferred_element_type=F32)
                    + p0b_ref[...], 0.0)
    h = jnp.maximum(jnp.dot(h, p1_ref[...], preferred_element_type=F32)
                    + p1b_ref[...], 0.0)
    out_ref[...] = (jnp.dot(h, ow_ref[...], preferred_element_type=F32)
                    + ob_ref[...])


def _head(sums, cnt, mol, args):
    return pl.pallas_call(
        _head_body,
        out_shape=jax.ShapeDtypeStruct((N_GRAPHS, 1), F32),
    )(sums, cnt, mol, *args)


def kernel(x, edge_index, edge_attr, batch_index, mol_features, params):
    ep = E_PAD - N_EDGES
    row_p = jnp.pad(edge_index[0], (0, ep))
    col2 = jnp.pad(edge_index[1], (0, ep)).reshape(NWORK, NWIN, W)
    eaT = jnp.pad(edge_attr.T, ((0, 0), (0, ep)))
    x_p = jnp.pad(x, ((0, N_PAD - N_NODES), (0, 0)))
    bi3 = jnp.pad(batch_index, (0, N_PAD - N_NODES),
                  constant_values=N_GRAPHS).reshape(NBLK, 1, NB)
    zn = jnp.zeros((N_PAD,), F32)
    zf = jnp.zeros((N_PAD, D), F32)

    convs = [params['conv0'], params['conv1'], params['conv2']]
    w1t = jnp.concatenate([cv['mlp_w1'] for cv in convs], axis=1).T
    b1c = jnp.concatenate([cv['mlp_b1'] for cv in convs])[:, None]
    w2t = jnp.zeros((8, 96), F32)
    b2c = jnp.zeros((8, 1), F32)
    for l, cv in enumerate(convs):
        w2t = w2t.at[l, 32 * l:32 * (l + 1)].set(cv['mlp_w2'][:, 0])
        b2c = b2c.at[l, 0].set(cv['mlp_b2'][0])

    e8 = _edge_mlp(eaT, w1t, b1c, w2t, b2c)
    e8f = e8.reshape(-1)
    degp = _build_deg_kernel()(col2, e8f, zn).reshape(NC * 3, N_PAD)
    dinv3, yt = _norm_xt0(degp, x_p, convs[0]['lin_w'])

    gls = [params['gcn_lin0'], params['gcn_lin1'], params['gcn_lin2']]
    for l in range(3):
        part = _build_agg_kernel()(yt, row_p, col2, e8[l], zf)
        bias = convs[l]['bias'][None, :]
        gw, gb = gls[l]['w'], gls[l]['b'][None, :]
        if l < 2:
            yt = _post(l, part, bias, gw, gb, convs[l + 1]['lin_w'], dinv3)
        else:
            y2 = _post_last(part, bias, gw, gb, dinv3)

    sums, cnt = _pool(bi3, y2)

    head_args = [
        params['mlp0']['w'], params['mlp0']['b'][None, :],
        params['mlp1']['w'], params['mlp1']['b'][None, :],
        params['mlp2']['w'], params['mlp2']['b'][None, :],
        params['pred0']['w'][:D], params['pred0']['w'][D:],
        params['pred0']['b'][None, :],
        params['pred1']['w'], params['pred1']['b'][None, :],
        params['out']['w'], params['out']['b'][None, :],
    ]
    return _head(sums, cnt, mol_features, head_args)
